# Initial kernel scaffold; baseline (speedup 1.0000x reference)
import jax
import jax.numpy as jnp
from jax.experimental import pallas as pl


def kernel(X, t):
    raise NotImplementedError("write your pallas kernel here")


# speedup vs baseline: 1.3949x; 1.3949x over previous
import functools

import jax
import jax.numpy as jnp
from jax import lax
from jax.experimental import pallas as pl
from jax.experimental.pallas import tpu as pltpu
from jax.experimental.pallas import tpu_sc as plsc

B, H, W, C = 8, 224, 224, 96
OUT_H, OUT_W = 224, 224
NPIX = B * OUT_H * OUT_W
PIX_PER_BATCH = OUT_H * OUT_W
NWORKERS = 32
PPW = NPIX // NWORKERS
CHUNK = 128
NCHUNKS = PPW // CHUNK
CC = C // 16
STEP = float(2.0 / 223.0)


def _bf16_round(v):
    u = lax.bitcast_convert_type(v, jnp.uint32)
    r = (u + jnp.uint32(0x7FFF) + ((u >> jnp.uint32(16)) & jnp.uint32(1)))
    r = r & jnp.uint32(0xFFFF0000)
    return lax.bitcast_convert_type(r, jnp.float32)


def _sc_body(x_hbm, coef_hbm, out_hbm,
             tv, ia_v, ib_v, ic_v, id_v, wv, ra, rb, rc, rd, ov, sem):
    cid = lax.axis_index("c")
    sid = lax.axis_index("s")
    wid = cid * 16 + sid
    base = wid * PPW
    batch = lax.div(base, PIX_PER_BATCH)
    inb0 = base - batch * PIX_PER_BATCH
    batch_off = batch * PIX_PER_BATCH

    pltpu.sync_copy(coef_hbm.at[batch], tv)
    ax = _bf16_round(tv[pl.ds(0, 16)])
    bx = _bf16_round(tv[pl.ds(16, 16)])
    cx = _bf16_round(tv[pl.ds(32, 16)])
    ay = _bf16_round(tv[pl.ds(48, 16)])
    by = _bf16_round(tv[pl.ds(64, 16)])
    cy = _bf16_round(tv[pl.ds(80, 16)])
    lane = lax.iota(jnp.int32, 16)

    def chunk_body(ch, carry):
        p0 = inb0 + ch * CHUNK
        for g in range(CHUNK // 16):
            p = p0 + g * 16 + lane
            i = lax.div(p, jnp.int32(OUT_W))
            j = p - i * OUT_W
            xc = _bf16_round(j.astype(jnp.float32) * STEP - 1.0)
            yc = _bf16_round(i.astype(jnp.float32) * STEP - 1.0)
            xs = ax * xc + bx * yc + cx
            ys = ay * xc + by * yc + cy
            xpix = (0.5 * (xs + 1.0)) * float(H)
            ypix = (0.5 * (ys + 1.0)) * float(W)
            x0 = xpix.astype(jnp.int32)
            y0 = ypix.astype(jnp.int32)
            x0c = jnp.clip(x0, 0, H - 1)
            x1c = jnp.clip(x0 + 1, 0, H - 1)
            y0c = jnp.clip(y0, 0, W - 1)
            y1c = jnp.clip(y0 + 1, 0, W - 1)
            x0f = x0c.astype(jnp.float32)
            x1f = x1c.astype(jnp.float32)
            y0f = y0c.astype(jnp.float32)
            y1f = y1c.astype(jnp.float32)
            wx0 = x1f - xpix
            wx1 = xpix - x0f
            wy0 = y1f - ypix
            wy1 = ypix - y0f
            row0 = batch_off + y0c * W
            row1 = batch_off + y1c * W
            sl = pl.ds(g * 16, 16)
            ia_v[sl] = row0 + x0c
            ib_v[sl] = row1 + x0c
            ic_v[sl] = row0 + x1c
            id_v[sl] = row1 + x1c
            wv[0, sl] = wx0 * wy0
            wv[1, sl] = wx0 * wy1
            wv[2, sl] = wx1 * wy0
            wv[3, sl] = wx1 * wy1

        da = pltpu.async_copy(x_hbm.at[ia_v], ra, sem)
        db = pltpu.async_copy(x_hbm.at[ib_v], rb, sem)
        dc = pltpu.async_copy(x_hbm.at[ic_v], rc, sem)
        dd = pltpu.async_copy(x_hbm.at[id_v], rd, sem)
        da.wait()
        db.wait()
        dc.wait()
        dd.wait()

        def group_body(g, c2):
            k0 = g * 16
            wa_g = wv[0, pl.ds(k0, 16)]
            wb_g = wv[1, pl.ds(k0, 16)]
            wc_g = wv[2, pl.ds(k0, 16)]
            wd_g = wv[3, pl.ds(k0, 16)]
            for l in range(16):
                k = k0 + l
                wa = wa_g[l]
                wb = wb_g[l]
                wc = wc_g[l]
                wd = wd_g[l]
                for cc in range(CC):
                    s2 = pl.ds(cc * 16, 16)
                    ov[k, s2] = (ra[k, s2] * wa + rb[k, s2] * wb
                                 + rc[k, s2] * wc + rd[k, s2] * wd)
            return c2

        lax.fori_loop(0, CHUNK // 16, group_body, 0)

        pltpu.sync_copy(ov, out_hbm.at[pl.ds(base + ch * CHUNK, CHUNK)])
        return carry

    lax.fori_loop(0, NCHUNKS, chunk_body, 0)


_sc_call = functools.partial(
    pl.kernel,
    mesh=plsc.VectorSubcoreMesh(core_axis_name="c", subcore_axis_name="s"),
    compiler_params=pltpu.CompilerParams(use_tc_tiling_on_sc=False),
    out_type=jax.ShapeDtypeStruct((NPIX, C), jnp.float32),
    scratch_types=[
        pltpu.VMEM((C,), jnp.float32),
        pltpu.VMEM((CHUNK,), jnp.int32),
        pltpu.VMEM((CHUNK,), jnp.int32),
        pltpu.VMEM((CHUNK,), jnp.int32),
        pltpu.VMEM((CHUNK,), jnp.int32),
        pltpu.VMEM((4, CHUNK), jnp.float32),
        pltpu.VMEM((CHUNK, C), jnp.float32),
        pltpu.VMEM((CHUNK, C), jnp.float32),
        pltpu.VMEM((CHUNK, C), jnp.float32),
        pltpu.VMEM((CHUNK, C), jnp.float32),
        pltpu.VMEM((CHUNK, C), jnp.float32),
        pltpu.SemaphoreType.DMA,
    ],
)(_sc_body)


def kernel(X, t):
    xf = X.reshape(NPIX, C).astype(jnp.float32)
    coef = jnp.repeat(t.astype(jnp.float32), 16, axis=1)
    out = _sc_call(xf, coef)
    return out.reshape(B, OUT_H, OUT_W, C)
